# Initial kernel scaffold; baseline (speedup 1.0000x reference)
import jax
import jax.numpy as jnp
from jax.experimental import pallas as pl


def kernel(input_ids, item_codes, centroids):
    raise NotImplementedError("write your pallas kernel here")


# speedup vs baseline: 9.6979x; 9.6979x over previous
import functools

import jax
import jax.numpy as jnp
from jax import lax
from jax.experimental import pallas as pl
from jax.experimental.pallas import tpu as pltpu
from jax.experimental.pallas import tpu_sc as plsc

_BATCH = 1024
_SEQ = 200
_PQ_M = 8
_VALS = 256
_SUB = 16

_N = _BATCH * _SEQ
_NC, _NS = 2, 16
_NW = _NC * _NS
_TOK_W = _N // _NW
_C = 256
_NCHUNK = _TOK_W // _C
_IDROWS = _C // 128
_FROWS = _C * _PQ_M // 128


def _body(ids_hbm, codes_hbm, cent_hbm, out_hbm, ids_v, codes_v, fidx_v,
          out_v, sem):
    wid = lax.axis_index("s") * _NC + lax.axis_index("c")
    lane = jnp.arange(16, dtype=jnp.int32)
    row_pat = lane >> 3
    col_pat = lane & 7
    off_pat = col_pat << 8

    def chunk(c, _):
        tok0 = wid * _TOK_W + c * _C
        irow0 = tok0 // 128
        orow0 = tok0 // 16

        pltpu.sync_copy(ids_hbm.at[pl.ds(irow0, _IDROWS)], ids_v)

        code_copies = [
            pltpu.async_copy(
                codes_hbm.at[ids_v.at[j]],
                codes_v.at[pl.ds(j * 128, 128)],
                sem,
            )
            for j in range(_IDROWS)
        ]
        for cp in code_copies:
            cp.wait()

        def fidx_row(j, _):
            for k in range(8):
                g = j * 8 + k
                rows = (2 * g) + row_pat
                code16 = plsc.load_gather(codes_v, [rows, col_pat])
                fidx_v[j, pl.ds(k * 16, 16)] = code16 + off_pat
            return _

        lax.fori_loop(0, _FROWS, fidx_row, None, unroll=True)

        cent_copies = [
            pltpu.async_copy(
                cent_hbm.at[fidx_v.at[j]],
                out_v.at[j],
                sem,
            )
            for j in range(_FROWS)
        ]
        for cp in cent_copies:
            cp.wait()

        pltpu.sync_copy(out_v, out_hbm.at[pl.ds(orow0, _FROWS)])
        return _

    lax.fori_loop(0, _NCHUNK, chunk, None)


@functools.partial(jax.jit, static_argnames=())
def kernel(input_ids, item_codes, centroids):
    ids2 = input_ids.reshape(_N // 128, 128)
    cent = centroids.reshape(_PQ_M * _VALS, _SUB)
    run = pl.kernel(
        _body,
        out_type=jax.ShapeDtypeStruct((_N * _PQ_M // 128, 128, _SUB),
                                      jnp.float32),
        mesh=plsc.VectorSubcoreMesh(
            core_axis_name="c", subcore_axis_name="s",
            num_cores=_NC, num_subcores=_NS,
        ),
        scratch_types=[
            pltpu.VMEM((_IDROWS, 128), jnp.int32),
            pltpu.VMEM((_C, _PQ_M), jnp.int32),
            pltpu.VMEM((_FROWS, 128), jnp.int32),
            pltpu.VMEM((_FROWS, 128, _SUB), jnp.float32),
            pltpu.SemaphoreType.DMA,
        ],
        compiler_params=pltpu.CompilerParams(use_tc_tiling_on_sc=False,
                                             needs_layout_passes=False),
    )
    out = run(ids2, item_codes, cent)
    return out.reshape(_BATCH, _SEQ, _PQ_M * _SUB)


# speedup vs baseline: 9.8579x; 1.0165x over previous
import functools

import jax
import jax.numpy as jnp
from jax import lax
from jax.experimental import pallas as pl
from jax.experimental.pallas import tpu as pltpu
from jax.experimental.pallas import tpu_sc as plsc

_BATCH = 1024
_SEQ = 200
_PQ_M = 8
_VALS = 256
_SUB = 16

_N = _BATCH * _SEQ
_NC, _NS = 2, 16
_NW = _NC * _NS
_TOK_W = _N // _NW
_C = 256
_NCHUNK = _TOK_W // _C
_IDROWS = _C // 128
_FROWS = _C * _PQ_M // 128


def _body(ids_hbm, codes_hbm, cent_hbm, out_hbm, ids_v, codes_v, fidx_v,
          out_v, sem_c, sem_g, sem_w):
    wid = lax.axis_index("s") * _NC + lax.axis_index("c")
    base_tok = wid * _TOK_W
    lane = jnp.arange(16, dtype=jnp.int32)
    row_pat = lane >> 3
    col_pat = lane & 7
    off_pat = col_pat << 8

    def prefetch(c, buf):
        tok0 = base_tok + c * _C
        pltpu.sync_copy(ids_hbm.at[pl.ds(tok0, _C)], ids_v.at[buf])
        for j in range(_IDROWS):
            pltpu.async_copy(
                codes_hbm.at[ids_v.at[buf, pl.ds(j * 128, 128)]],
                codes_v.at[buf, pl.ds(j * 128, 128)],
                sem_c,
            )

    def wait_codes():
        for j in range(_IDROWS):
            pltpu.make_async_copy(
                codes_hbm.at[ids_v.at[0, pl.ds(0, 128)]],
                codes_v.at[0, pl.ds(0, 128)],
                sem_c,
            ).wait()

    prefetch(0, 0)

    def chunk(c, _):
        buf = c & 1
        tok0 = base_tok + c * _C
        orow0 = tok0 // 16

        wait_codes()

        @pl.when(c + 1 < _NCHUNK)
        def _():
            prefetch(c + 1, 1 - buf)

        @pl.when(c >= 2)
        def _():
            pltpu.make_async_copy(
                out_v.at[buf],
                out_hbm.at[pl.ds(0, _FROWS)],
                sem_w.at[buf],
            ).wait()

        def fidx_row(j, _):
            for k in range(8):
                g = j * 8 + k
                rows = (2 * g) + row_pat
                code16 = plsc.load_gather(
                    codes_v.at[buf], [rows, col_pat])
                fidx_v[j, pl.ds(k * 16, 16)] = code16 + off_pat
            pltpu.async_copy(
                cent_hbm.at[fidx_v.at[j]],
                out_v.at[buf, j],
                sem_g,
            )
            return _

        lax.fori_loop(0, _FROWS, fidx_row, None, unroll=True)

        for j in range(_FROWS):
            pltpu.make_async_copy(
                cent_hbm.at[fidx_v.at[0]],
                out_v.at[0, 0],
                sem_g,
            ).wait()

        pltpu.async_copy(
            out_v.at[buf],
            out_hbm.at[pl.ds(orow0, _FROWS)],
            sem_w.at[buf],
        )
        return _

    lax.fori_loop(0, _NCHUNK, chunk, None)

    for b in range(2):
        pltpu.make_async_copy(
            out_v.at[b],
            out_hbm.at[pl.ds(0, _FROWS)],
            sem_w.at[b],
        ).wait()


@functools.partial(jax.jit, static_argnames=())
def kernel(input_ids, item_codes, centroids):
    ids1 = input_ids.reshape(_N)
    cent = centroids.reshape(_PQ_M * _VALS, _SUB)
    run = pl.kernel(
        _body,
        out_type=jax.ShapeDtypeStruct((_N * _PQ_M // 128, 128, _SUB),
                                      jnp.float32),
        mesh=plsc.VectorSubcoreMesh(
            core_axis_name="c", subcore_axis_name="s",
            num_cores=_NC, num_subcores=_NS,
        ),
        scratch_types=[
            pltpu.VMEM((2, _C), jnp.int32),
            pltpu.VMEM((2, _C, _PQ_M), jnp.int32),
            pltpu.VMEM((_FROWS, 128), jnp.int32),
            pltpu.VMEM((2, _FROWS, 128, _SUB), jnp.float32),
            pltpu.SemaphoreType.DMA,
            pltpu.SemaphoreType.DMA,
            pltpu.SemaphoreType.DMA((2,)),
        ],
        compiler_params=pltpu.CompilerParams(use_tc_tiling_on_sc=False,
                                             needs_layout_passes=False),
    )
    out = run(ids1, item_codes, cent)
    return out.reshape(_BATCH, _SEQ, _PQ_M * _SUB)
